# baseline (speedup 1.0000x reference)
import functools

import jax
import jax.numpy as jnp
from jax import lax
from jax.experimental import pallas as pl
from jax.experimental.pallas import tpu as pltpu
from jax.experimental.pallas import tpu_sc as plsc

B = 4096
LA = 50
LP = 20
LN = 200
D = 64
L = 220
GAMMA = 0.5

NC = 2
NS = 16
NW = NC * NS
BPW = B // NW

IW = 384
NA = 128
NT = 240
WW = 128


def _mask_body(pos_l_ref, neg_l_ref, ali_ref, alu_ref, mask_ref, tgt_ref, w_ref):
    iota = lax.broadcasted_iota(jnp.int32, (B, L), 1)
    pos_l = pos_l_ref[...]
    neg_l = neg_l_ref[...]
    mp = jnp.where(iota < pos_l, 1, 0)
    mn = jnp.where((iota - LP) < neg_l, 1, 0)
    m = jnp.where(iota < LP, mp, mn)
    mask_ref[...] = m
    tgt_ref[...] = jnp.where(iota < LP, m, 0)
    iw = lax.broadcasted_iota(jnp.int32, (B, WW), 1)
    lens = jnp.where(iw < 64, ali_ref[...], alu_ref[...])
    j = jnp.where(iw < 64, iw, iw - 64)
    w_ref[...] = jnp.where(
        j < lens, GAMMA / lens.astype(jnp.float32), jnp.float32(0.0))


_mask_call = pl.pallas_call(
    _mask_body,
    out_shape=(
        jax.ShapeDtypeStruct((B, L), jnp.int32),
        jax.ShapeDtypeStruct((B, L), jnp.int32),
        jax.ShapeDtypeStruct((B, WW), jnp.float32),
    ),
)


def _row_f32(ref, j, cols):
    out = []
    for c in cols:
        v = plsc.bitcast(ref[j, pl.ds(c, 32)], jnp.int32)
        lo = plsc.bitcast(jnp.left_shift(v, 16), jnp.float32)
        hi = plsc.bitcast(jnp.bitwise_and(v, jnp.int32(-65536)), jnp.float32)
        out += [lo, hi]
    return out


def _sc_body(idx_hbm, w_hbm, attr_t, user_t, item_t, out_t, out_hbm,
             idxb, wb, arows, urow, irow, trows, logv0, logv1,
             isem0, isem1, gsem0, gsem1, osem0, osem1):
    logv = (logv0, logv1)
    wid = lax.axis_index("s") * NC + lax.axis_index("c")
    base = wid * BPW
    isem = (isem0, isem1)
    gsem = (gsem0, gsem1)
    osem = (osem0, osem1)

    def idx_cps(j, r):
        return [
            pltpu.make_async_copy(idx_hbm.at[base + j], idxb.at[r], isem[r]),
            pltpu.make_async_copy(w_hbm.at[base + j], wb.at[r], isem[r]),
        ]

    def gather_cps(r):
        ib = idxb.at[r]
        return [
            pltpu.make_async_copy(user_t.at[ib.at[pl.ds(0, 1)]],
                                  urow.at[r], gsem[r]),
            pltpu.make_async_copy(item_t.at[ib.at[pl.ds(8, 1)]],
                                  irow.at[r], gsem[r]),
            pltpu.make_async_copy(attr_t.at[ib.at[pl.ds(16, NA)]],
                                  arows.at[r], gsem[r]),
            pltpu.make_async_copy(out_t.at[ib.at[pl.ds(144, 128)]],
                                  trows.at[r, pl.ds(0, 128)], gsem[r]),
            pltpu.make_async_copy(out_t.at[ib.at[pl.ds(272, 112)]],
                                  trows.at[r, pl.ds(128, 112)], gsem[r]),
        ]

    def out_cp(j, r):
        return pltpu.make_async_copy(logv[r].at[pl.ds(8, L)],
                                     out_hbm.at[base + j], osem[r])

    def compute(j, r):
        @pl.when(j >= 2)
        def _():
            out_cp(j - 2, r).wait()

        zero = jnp.zeros((16,), jnp.float32)
        acc_i = [[zero] * 4 for _ in range(4)]
        acc_u = [[zero] * 4 for _ in range(4)]
        ar = arows.at[r]
        for c in range(4):
            wci = wb[r, pl.ds(c * 16, 16)]
            wcu = wb[r, pl.ds(64 + c * 16, 16)]
            for t16 in range(16):
                jj = c * 16 + t16
                sp = jnp.full((16,), t16, jnp.int32)
                wi = jnp.take_along_axis(wci, sp, axis=0,
                                         mode="promise_in_bounds")
                wu = jnp.take_along_axis(wcu, sp, axis=0,
                                         mode="promise_in_bounds")
                p = t16 % 4
                ri = _row_f32(ar, jj, (0, 32))
                ru = _row_f32(ar, jj + 64, (0, 32))
                for k in range(4):
                    acc_i[p][k] = acc_i[p][k] + ri[k] * wi
                    acc_u[p][k] = acc_u[p][k] + ru[k] * wu
        half = jnp.float32(1.0 - GAMMA)
        urr = _row_f32(urow.at[r], 0, (0, 32))
        irr = _row_f32(irow.at[r], 0, (0, 32))
        w8 = []
        for k in range(4):
            w8.append(half * urr[k]
                      + (acc_u[0][k] + acc_u[1][k])
                      + (acc_u[2][k] + acc_u[3][k]))
        for k in range(4):
            w8.append(half * irr[k]
                      + (acc_i[0][k] + acc_i[1][k])
                      + (acc_i[2][k] + acc_i[3][k]))

        iota16 = lax.iota(jnp.int32, 16)
        xors = [jnp.bitwise_xor(iota16, jnp.int32(dd)) for dd in (8, 4, 2, 1)]

        def tgt_body(g, carry2):
            gb = g * 16
            sub = trows.at[r, pl.ds(gb, 16)]
            lvec = jnp.zeros((16,), jnp.float32)
            for t16 in range(16):
                rv = _row_f32(sub, t16, (0, 32, 64, 96))
                accs4 = [rv[2 * k] * w8[2 * k] + rv[2 * k + 1] * w8[2 * k + 1]
                         for k in range(4)]
                acc = (accs4[0] + accs4[1]) + (accs4[2] + accs4[3])
                for xi in xors:
                    acc = acc + jnp.take_along_axis(
                        acc, xi, axis=0, mode="promise_in_bounds")
                lvec = jnp.where(iota16 == t16, acc, lvec)
            logv[r][pl.ds(gb, 16)] = lvec
            return carry2

        lax.fori_loop(0, NT // 16, tgt_body, 0)
        out_cp(j, r).start()

    pro = idx_cps(0, 0)
    for c in pro:
        c.start()
    for c in pro:
        c.wait()
    for c in idx_cps(1, 1):
        c.start()
    for c in gather_cps(0):
        c.start()

    def qbody(q, carry):
        for s in range(2):
            j = 2 * q + s
            r = s

            @pl.when(j + 1 <= BPW - 1)
            def _(j=j, r=r):
                for c in idx_cps(j + 1, 1 - r):
                    c.wait()
                for c in gather_cps(1 - r):
                    c.start()

            for c in gather_cps(r):
                c.wait()

            @pl.when(j + 2 <= BPW - 1)
            def _(j=j, r=r):
                for c in idx_cps(j + 2, r):
                    c.start()

            compute(j, r)
        return carry

    lax.fori_loop(0, BPW // 2, qbody, 0)
    out_cp(BPW - 2, 0).wait()
    out_cp(BPW - 1, 1).wait()


_sc_call = functools.partial(
    pl.kernel,
    out_type=jax.ShapeDtypeStruct((B, L), jnp.float32),
    mesh=plsc.VectorSubcoreMesh(core_axis_name="c", subcore_axis_name="s"),
    compiler_params=pltpu.CompilerParams(use_tc_tiling_on_sc=False,
                                         needs_layout_passes=False),
    scratch_types=[
        pltpu.VMEM((2, IW), jnp.int32),
        pltpu.VMEM((2, WW), jnp.float32),
        pltpu.VMEM((2, NA, D), jnp.bfloat16),
        pltpu.VMEM((2, 1, D), jnp.bfloat16),
        pltpu.VMEM((2, 1, D), jnp.bfloat16),
        pltpu.VMEM((2, NT, 2 * D), jnp.bfloat16),
        pltpu.VMEM((NT,), jnp.float32),
        pltpu.VMEM((NT,), jnp.float32),
        pltpu.SemaphoreType.DMA,
        pltpu.SemaphoreType.DMA,
        pltpu.SemaphoreType.DMA,
        pltpu.SemaphoreType.DMA,
        pltpu.SemaphoreType.DMA,
        pltpu.SemaphoreType.DMA,
    ],
)(_sc_body)


def kernel(attr_item, attr_tf_item, attr_lens_item, item_ids, attr_user,
           attr_tf_user, attr_lens_user, user_ids, pos_targets, pos_lens,
           neg_targets, neg_lens, attr_table, user_table, item_table,
           out_user_table, out_item_table):
    i32 = jnp.int32
    bf = jnp.bfloat16
    z = lambda n: jnp.zeros((B, n), i32)
    packed_idx = jnp.concatenate([
        user_ids[:, None].astype(i32), z(7),
        item_ids[:, None].astype(i32), z(7),
        attr_item.astype(i32), z(14),
        attr_user.astype(i32), z(14),
        z(8),
        pos_targets.astype(i32),
        neg_targets.astype(i32), z(12),
    ], axis=1)

    mask_i, new_targets, packed_w = _mask_call(
        pos_lens[:, None].astype(i32), neg_lens[:, None].astype(i32),
        attr_lens_item[:, None].astype(i32),
        attr_lens_user[:, None].astype(i32))

    out_comb = jnp.concatenate([out_user_table, out_item_table],
                               axis=1).astype(bf)
    logits = _sc_call(packed_idx, packed_w, attr_table.astype(bf),
                      user_table.astype(bf), item_table.astype(bf), out_comb)
    return (logits, mask_i.astype(jnp.bool_), new_targets)


# speedup vs baseline: 2.0012x; 2.0012x over previous
import functools

import jax
import jax.numpy as jnp
from jax import lax
from jax.experimental import pallas as pl
from jax.experimental.pallas import tpu as pltpu
from jax.experimental.pallas import tpu_sc as plsc

B = 4096
LA = 50
LP = 20
LN = 200
D = 64
L = 220
GAMMA = 0.5

NC = 2
NS = 16
NW = NC * NS
BPW = B // NW

IW = 352
NA = 112
NT = 224
WW = 128


def _mask_body(pos_l_ref, neg_l_ref, ali_ref, alu_ref, mask_ref, tgt_ref, w_ref):
    iota = lax.broadcasted_iota(jnp.int32, (B, L), 1)
    pos_l = pos_l_ref[...]
    neg_l = neg_l_ref[...]
    mp = jnp.where(iota < pos_l, 1, 0)
    mn = jnp.where((iota - LP) < neg_l, 1, 0)
    m = jnp.where(iota < LP, mp, mn)
    mask_ref[...] = m
    tgt_ref[...] = jnp.where(iota < LP, m, 0)
    iw = lax.broadcasted_iota(jnp.int32, (B, WW), 1)
    lens = jnp.where(iw < 64, ali_ref[...], alu_ref[...])
    j = jnp.where(iw < 64, iw, iw - 64)
    w_ref[...] = jnp.where(
        j < lens, GAMMA / lens.astype(jnp.float32), jnp.float32(0.0))


_mask_call = pl.pallas_call(
    _mask_body,
    out_shape=(
        jax.ShapeDtypeStruct((B, L), jnp.int32),
        jax.ShapeDtypeStruct((B, L), jnp.int32),
        jax.ShapeDtypeStruct((B, WW), jnp.float32),
    ),
)


def _row_f32(ref, j, cols):
    out = []
    for c in cols:
        v = plsc.bitcast(ref[j, pl.ds(c, 32)], jnp.int32)
        lo = plsc.bitcast(jnp.left_shift(v, 16), jnp.float32)
        hi = plsc.bitcast(jnp.bitwise_and(v, jnp.int32(-65536)), jnp.float32)
        out += [lo, hi]
    return out


def _sc_body(idx_hbm, w_hbm, attr_t, user_t, item_t, out_t, out_hbm,
             idxb, wb, arows, urow, irow, trows, logv0, logv1,
             isem0, isem1, gsem0, gsem1, osem0, osem1):
    logv = (logv0, logv1)
    wid = lax.axis_index("s") * NC + lax.axis_index("c")
    base = wid * BPW
    isem = (isem0, isem1)
    gsem = (gsem0, gsem1)
    osem = (osem0, osem1)

    def idx_cps(j, r):
        return [
            pltpu.make_async_copy(idx_hbm.at[base + j], idxb.at[r], isem[r]),
            pltpu.make_async_copy(w_hbm.at[base + j], wb.at[r], isem[r]),
        ]

    def gather_cps(r):
        ib = idxb.at[r]
        return [
            pltpu.make_async_copy(user_t.at[ib.at[pl.ds(0, 1)]],
                                  urow.at[r], gsem[r]),
            pltpu.make_async_copy(item_t.at[ib.at[pl.ds(8, 1)]],
                                  irow.at[r], gsem[r]),
            pltpu.make_async_copy(attr_t.at[ib.at[pl.ds(16, NA)]],
                                  arows.at[r], gsem[r]),
            pltpu.make_async_copy(out_t.at[ib.at[pl.ds(128, 128)]],
                                  trows.at[r, pl.ds(0, 128)], gsem[r]),
            pltpu.make_async_copy(out_t.at[ib.at[pl.ds(256, 96)]],
                                  trows.at[r, pl.ds(128, 96)], gsem[r]),
        ]

    def out_cp(j, r):
        return pltpu.make_async_copy(logv[r].at[pl.ds(0, L)],
                                     out_hbm.at[base + j], osem[r])

    def compute(j, r, fire_next):
        @pl.when(j >= 2)
        def _():
            out_cp(j - 2, r).wait()

        zero = jnp.zeros((16,), jnp.float32)
        acc_i = [[zero] * 4 for _ in range(4)]
        acc_u = [[zero] * 4 for _ in range(4)]
        ar = arows.at[r]
        for c in range(4):
            wci = wb[r, pl.ds(c * 16, 16)]
            wcu = wb[r, pl.ds(64 + c * 16, 16)]
            for t16 in range(16 if c < 3 else 8):
                jj = c * 16 + t16
                sp = jnp.full((16,), t16, jnp.int32)
                wi = jnp.take_along_axis(wci, sp, axis=0,
                                         mode="promise_in_bounds")
                wu = jnp.take_along_axis(wcu, sp, axis=0,
                                         mode="promise_in_bounds")
                p = t16 % 4
                ri = _row_f32(ar, jj, (0, 32))
                ru = _row_f32(ar, jj + 56, (0, 32))
                for k in range(4):
                    acc_i[p][k] = acc_i[p][k] + ri[k] * wi
                    acc_u[p][k] = acc_u[p][k] + ru[k] * wu
        half = jnp.float32(1.0 - GAMMA)
        urr = _row_f32(urow.at[r], 0, (0, 32))
        irr = _row_f32(irow.at[r], 0, (0, 32))
        w8 = []
        for k in range(4):
            w8.append(half * urr[k]
                      + (acc_u[0][k] + acc_u[1][k])
                      + (acc_u[2][k] + acc_u[3][k]))
        for k in range(4):
            w8.append(half * irr[k]
                      + (acc_i[0][k] + acc_i[1][k])
                      + (acc_i[2][k] + acc_i[3][k]))

        fire_next()

        iota16 = lax.iota(jnp.int32, 16)
        xors = [jnp.bitwise_xor(iota16, jnp.int32(dd)) for dd in (8, 4, 2, 1)]

        def tgt_group(gb, n16):
            sub = trows.at[r, pl.ds(gb, 16)]
            lvec = jnp.zeros((16,), jnp.float32)
            for t16 in range(n16):
                rv = _row_f32(sub, t16, (0, 32, 64, 96))
                accs4 = [rv[2 * k] * w8[2 * k] + rv[2 * k + 1] * w8[2 * k + 1]
                         for k in range(4)]
                acc = (accs4[0] + accs4[1]) + (accs4[2] + accs4[3])
                for xi in xors:
                    acc = acc + jnp.take_along_axis(
                        acc, xi, axis=0, mode="promise_in_bounds")
                lvec = jnp.where(iota16 == t16, acc, lvec)
            logv[r][pl.ds(gb, 16)] = lvec

        def tgt_body(g, carry2):
            tgt_group(g * 16, 16)
            return carry2

        lax.fori_loop(0, 13, tgt_body, 0)
        tgt_group(208, 12)
        out_cp(j, r).start()

    pro = idx_cps(0, 0)
    for c in pro:
        c.start()
    for c in pro:
        c.wait()
    for c in idx_cps(1, 1):
        c.start()
    for c in gather_cps(0):
        c.start()

    def qbody(q, carry):
        for s in range(2):
            j = 2 * q + s
            r = s

            @pl.when(j + 1 <= BPW - 1)
            def _(j=j, r=r):
                for c in idx_cps(j + 1, 1 - r):
                    c.wait()
                for c in gather_cps(1 - r):
                    c.start()

            for c in gather_cps(r):
                c.wait()

            def fire_next(j=j, r=r):
                @pl.when(j + 2 <= BPW - 1)
                def _():
                    for c in idx_cps(j + 2, r):
                        c.start()

            compute(j, r, fire_next)
        return carry

    lax.fori_loop(0, BPW // 2, qbody, 0)
    out_cp(BPW - 2, 0).wait()
    out_cp(BPW - 1, 1).wait()


_sc_call = functools.partial(
    pl.kernel,
    out_type=jax.ShapeDtypeStruct((B, L), jnp.float32),
    mesh=plsc.VectorSubcoreMesh(core_axis_name="c", subcore_axis_name="s"),
    compiler_params=pltpu.CompilerParams(use_tc_tiling_on_sc=False,
                                         needs_layout_passes=False),
    scratch_types=[
        pltpu.VMEM((2, IW), jnp.int32),
        pltpu.VMEM((2, WW), jnp.float32),
        pltpu.VMEM((2, NA, D), jnp.bfloat16),
        pltpu.VMEM((2, 1, D), jnp.bfloat16),
        pltpu.VMEM((2, 1, D), jnp.bfloat16),
        pltpu.VMEM((2, NT, 2 * D), jnp.bfloat16),
        pltpu.VMEM((NT,), jnp.float32),
        pltpu.VMEM((NT,), jnp.float32),
        pltpu.SemaphoreType.DMA,
        pltpu.SemaphoreType.DMA,
        pltpu.SemaphoreType.DMA,
        pltpu.SemaphoreType.DMA,
        pltpu.SemaphoreType.DMA,
        pltpu.SemaphoreType.DMA,
    ],
)(_sc_body)


def kernel(attr_item, attr_tf_item, attr_lens_item, item_ids, attr_user,
           attr_tf_user, attr_lens_user, user_ids, pos_targets, pos_lens,
           neg_targets, neg_lens, attr_table, user_table, item_table,
           out_user_table, out_item_table):
    i32 = jnp.int32
    bf = jnp.bfloat16
    z = lambda n: jnp.zeros((B, n), i32)
    packed_idx = jnp.concatenate([
        user_ids[:, None].astype(i32), z(7),
        item_ids[:, None].astype(i32), z(7),
        attr_item.astype(i32), z(6),
        attr_user.astype(i32), z(6),
        pos_targets.astype(i32),
        neg_targets.astype(i32), z(4),
    ], axis=1)

    mask_i, new_targets, packed_w = _mask_call(
        pos_lens[:, None].astype(i32), neg_lens[:, None].astype(i32),
        attr_lens_item[:, None].astype(i32),
        attr_lens_user[:, None].astype(i32))

    out_comb = jnp.concatenate([out_user_table, out_item_table],
                               axis=1).astype(bf)
    logits = _sc_call(packed_idx, packed_w, attr_table.astype(bf),
                      user_table.astype(bf), item_table.astype(bf), out_comb)
    return (logits, mask_i.astype(jnp.bool_), new_targets)
